# baseline (speedup 1.0000x reference)
import functools

import jax
import jax.numpy as jnp
from jax import lax
from jax.experimental import pallas as pl
from jax.experimental.pallas import tpu as pltpu
from jax.experimental.pallas import tpu_sc as plsc

N_NODES = 10000
D_FEAT = 128
HALF = D_FEAT // 2
N_EDGES = 320000

N_TILES = 16
CHUNK = 128
N_CHUNKS = N_EDGES // CHUNK
CHUNKS_PER_TILE = N_CHUNKS // N_TILES
EDGES_PER_TILE = CHUNKS_PER_TILE * CHUNK
BLK = 3
BLK_E = BLK * CHUNK
N_BLKS = CHUNKS_PER_TILE // BLK
EXTRA_TILES = N_CHUNKS - N_TILES * CHUNKS_PER_TILE
EXTRA_BASE = N_TILES * EDGES_PER_TILE
N_PAD = 10240
ROWS_PER_TILE = N_PAD // N_TILES
WCHUNK = 128
N_WCHUNKS = ROWS_PER_TILE // WCHUNK
LANES = 16

_mesh = plsc.VectorSubcoreMesh(
    core_axis_name="c", subcore_axis_name="s", num_cores=2
)

_half = jax.ShapeDtypeStruct((N_PAD, HALF), jnp.float32)


@functools.partial(
    pl.kernel,
    out_type=(_half,) * 6,
    mesh=_mesh,
    compiler_params=pltpu.CompilerParams(use_tc_tiling_on_sc=False),
    scratch_types=[
        pltpu.VMEM_SHARED((N_PAD, HALF), jnp.float32),
        [pltpu.VMEM((BLK_E, HALF), jnp.float32)] * 3,
        [pltpu.VMEM((BLK_E,), jnp.int32)] * 6,
        [pltpu.VMEM((BLK_E,), jnp.int32)] * 6,
        [pltpu.SemaphoreType.DMA] * 6,
        pltpu.VMEM((WCHUNK, HALF), jnp.float32),
        [pltpu.SemaphoreType.DMA] * 3,
        [pltpu.SemaphoreType.DMA] * 3,
    ],
)
def _conv3(x_lo, x_hi, src, dst,
           h1_lo, h1_hi, h2_lo, h2_hi, o_lo, o_hi,
           acc, rows, idx_s, idx_d, isem, zbuf, gsem, ssem):
    cid = lax.axis_index("c")
    wid = lax.axis_index("s")
    ebase = wid * EDGES_PER_TILE
    rbase = wid * ROWS_PER_TILE

    zeros = jnp.zeros((LANES,), jnp.float32)

    def layer(src_buf, dst_buf, relu):
        def fire_idx(s, blk):
            off = ebase + jnp.minimum(blk, N_BLKS - 1) * BLK_E
            pltpu.async_copy(src.at[pl.ds(off, BLK_E)], idx_s[s], isem[s])
            pltpu.async_copy(dst.at[pl.ds(off, BLK_E)], idx_d[s], isem[s])

        def wait_idx(s):
            pltpu.make_async_copy(
                src.at[pl.ds(0, BLK_E)], idx_s[s], isem[s]).wait()
            pltpu.make_async_copy(
                dst.at[pl.ds(0, BLK_E)], idx_d[s], isem[s]).wait()

        def launch_g(r, s):
            wait_idx(s)
            for k in range(BLK):
                sl = pl.ds(k * CHUNK, CHUNK)
                pltpu.async_copy(
                    src_buf.at[idx_s[s].at[sl]], rows[r].at[sl], gsem[r])

        def wait_g(r, s):
            for k in range(BLK):
                sl = pl.ds(k * CHUNK, CHUNK)
                pltpu.make_async_copy(
                    src_buf.at[idx_s[s].at[sl]], rows[r].at[sl], gsem[r]).wait()

        def fire_s(r, s):
            for k in range(BLK):
                sl = pl.ds(k * CHUNK, CHUNK)
                pltpu.async_copy(
                    rows[r].at[sl], acc.at[idx_d[s].at[sl]], ssem[r], add=True)

        def drain_s(r, s):
            for k in range(BLK):
                sl = pl.ds(k * CHUNK, CHUNK)
                pltpu.make_async_copy(
                    rows[r].at[sl], acc.at[idx_d[s].at[sl]], ssem[r]).wait()

        def step(b_expr, bm3, bm6, first=False, launch=True, fire=True):
            wait_g(bm3, bm6)
            fire_s(bm3, bm6)
            if not first:
                drain_s((bm3 + 2) % 3, (bm6 + 5) % 6)
            if launch:
                launch_g((bm3 + 2) % 3, (bm6 + 2) % 6)
            if fire:
                fire_idx((bm6 + 4) % 6, b_expr + 4)

        @pl.when(wid < EXTRA_TILES)
        def _():
            off = EXTRA_BASE + wid * CHUNK
            csl = pl.ds(0, CHUNK)
            pltpu.sync_copy(src.at[pl.ds(off, CHUNK)], idx_s[0].at[csl])
            pltpu.sync_copy(dst.at[pl.ds(off, CHUNK)], idx_d[0].at[csl])
            pltpu.sync_copy(src_buf.at[idx_s[0].at[csl]], rows[0].at[csl])
            pltpu.sync_copy(rows[0].at[csl], acc.at[idx_d[0].at[csl]], add=True)

        for s in range(4):
            fire_idx(s, s)
        launch_g(0, 0)
        launch_g(1, 1)
        step(0, 0, 0, first=True)
        step(1, 1, 1)

        @pl.loop(0, (N_BLKS - 4) // 6)
        def _(t):
            b0 = 6 * t + 2
            for p in range(6):
                step(b0 + p, (2 + p) % 3, (2 + p) % 6)

        step(N_BLKS - 2, (N_BLKS - 2) % 3, (N_BLKS - 2) % 6,
             launch=False, fire=False)
        step(N_BLKS - 1, (N_BLKS - 1) % 3, (N_BLKS - 1) % 6,
             launch=False, fire=False)
        drain_s((N_BLKS - 1) % 3, (N_BLKS - 1) % 6)
        wait_idx(4)
        wait_idx(5)
        plsc.subcore_barrier()

        zdescs = []

        def restore_zero(k):
            zdescs.append(pltpu.async_copy(
                zbuf, acc.at[pl.ds(rbase + k * WCHUNK, WCHUNK)], ssem[0]))

        if relu:
            r1sl = pl.ds(0, 256)
            d0 = pltpu.async_copy(acc.at[pl.ds(rbase, BLK_E)], rows[0], gsem[0])
            d1 = pltpu.async_copy(
                acc.at[pl.ds(rbase + BLK_E, 256)], rows[1].at[r1sl], gsem[1])
            d0.wait()
            for k in range(3):
                restore_zero(k)

            @pl.loop(0, BLK_E)
            def _(r):
                for c in range(HALF // LANES):
                    v = rows[0][r, pl.ds(c * LANES, LANES)]
                    rows[0][r, pl.ds(c * LANES, LANES)] = jnp.maximum(v, 0.0)

            w0 = pltpu.async_copy(
                rows[0], dst_buf.at[pl.ds(rbase, BLK_E)], ssem[1])
            d1.wait()
            restore_zero(3)
            restore_zero(4)

            @pl.loop(0, 256)
            def _(r):
                for c in range(HALF // LANES):
                    v = rows[1][r, pl.ds(c * LANES, LANES)]
                    rows[1][r, pl.ds(c * LANES, LANES)] = jnp.maximum(v, 0.0)

            w1 = pltpu.async_copy(
                rows[1].at[r1sl],
                dst_buf.at[pl.ds(rbase + BLK_E, 256)], ssem[1])
            w0.wait()
            w1.wait()
        else:
            w0 = pltpu.async_copy(
                acc.at[pl.ds(rbase, ROWS_PER_TILE)],
                dst_buf.at[pl.ds(rbase, ROWS_PER_TILE)], ssem[1])
            w0.wait()
            for k in range(N_WCHUNKS):
                restore_zero(k)
        for d in zdescs:
            d.wait()
        plsc.subcore_barrier()

    @pl.loop(0, WCHUNK)
    def _(r):
        for c in range(HALF // LANES):
            zbuf[r, pl.ds(c * LANES, LANES)] = zeros

    for k in range(N_WCHUNKS):
        pltpu.sync_copy(zbuf, acc.at[pl.ds(rbase + k * WCHUNK, WCHUNK)])
    plsc.subcore_barrier()

    @pl.when(cid == 0)
    def _():
        layer(x_lo, h1_lo, True)
        layer(h1_lo, h2_lo, False)
        layer(h2_lo, o_lo, False)

    @pl.when(cid == 1)
    def _():
        layer(x_hi, h1_hi, True)
        layer(h1_hi, h2_hi, False)
        layer(h2_hi, o_hi, False)


def kernel(x, edge_index):
    src = edge_index[0].astype(jnp.int32)
    dst = edge_index[1].astype(jnp.int32)
    x_lo = x[:, :HALF]
    x_hi = x[:, HALF:]
    *_, o_lo, o_hi = _conv3(x_lo, x_hi, src, dst)
    return jnp.concatenate([o_lo[:N_NODES], o_hi[:N_NODES]], axis=1)


# speedup vs baseline: 1.0528x; 1.0528x over previous
import functools

import jax
import jax.numpy as jnp
from jax import lax
from jax.experimental import pallas as pl
from jax.experimental.pallas import tpu as pltpu
from jax.experimental.pallas import tpu_sc as plsc

N_NODES = 10000
D_FEAT = 128
HALF = D_FEAT // 2
N_EDGES = 320000

N_TILES = 16
CHUNK = 128
N_CHUNKS = N_EDGES // CHUNK
CHUNKS_PER_TILE = N_CHUNKS // N_TILES
EDGES_PER_TILE = CHUNKS_PER_TILE * CHUNK
BLK = 4
BLK_E = BLK * CHUNK
N_BLKS = CHUNKS_PER_TILE // BLK
EXTRA_TILES = N_CHUNKS - N_TILES * CHUNKS_PER_TILE
EXTRA_BASE = N_TILES * EDGES_PER_TILE
N_PAD = 10240
ROWS_PER_TILE = N_PAD // N_TILES
WCHUNK = 128
N_WCHUNKS = ROWS_PER_TILE // WCHUNK
LANES = 16

_mesh = plsc.VectorSubcoreMesh(
    core_axis_name="c", subcore_axis_name="s", num_cores=2
)

_half = jax.ShapeDtypeStruct((N_PAD, HALF), jnp.float32)


@functools.partial(
    pl.kernel,
    out_type=(_half,) * 6,
    mesh=_mesh,
    compiler_params=pltpu.CompilerParams(use_tc_tiling_on_sc=False),
    scratch_types=[
        pltpu.VMEM_SHARED((N_PAD, HALF), jnp.float32),
        pltpu.VMEM((BLK_E, HALF), jnp.float32),
        pltpu.VMEM((BLK_E, HALF), jnp.float32),
        [pltpu.VMEM((BLK_E,), jnp.int32)] * 4,
        [pltpu.VMEM((BLK_E,), jnp.int32)] * 4,
        [pltpu.SemaphoreType.DMA] * 4,
        pltpu.VMEM((WCHUNK, HALF), jnp.float32),
        pltpu.SemaphoreType.DMA,
        pltpu.SemaphoreType.DMA,
        pltpu.SemaphoreType.DMA,
        pltpu.SemaphoreType.DMA,
    ],
)
def _conv3(x_lo, x_hi, src, dst,
           h1_lo, h1_hi, h2_lo, h2_hi, o_lo, o_hi,
           acc, rows0, rows1, idx_s, idx_d, isem, zbuf,
           gsem0, gsem1, ssem0, ssem1):
    cid = lax.axis_index("c")
    wid = lax.axis_index("s")
    ebase = wid * EDGES_PER_TILE
    rbase = wid * ROWS_PER_TILE

    zeros = jnp.zeros((LANES,), jnp.float32)

    def layer(src_buf, dst_buf, relu):
        rbufs = ((rows0, gsem0, ssem0), (rows1, gsem1, ssem1))

        def fire_idx(s, blk):
            off = ebase + blk * BLK_E
            pltpu.async_copy(src.at[pl.ds(off, BLK_E)], idx_s[s], isem[s])
            pltpu.async_copy(dst.at[pl.ds(off, BLK_E)], idx_d[s], isem[s])

        def wait_idx(s):
            pltpu.make_async_copy(
                src.at[pl.ds(0, BLK_E)], idx_s[s], isem[s]).wait()
            pltpu.make_async_copy(
                dst.at[pl.ds(0, BLK_E)], idx_d[s], isem[s]).wait()

        def launch_g(b, s):
            r_ref, gsem, _ = rbufs[b]
            wait_idx(s)
            for k in range(BLK):
                sl = pl.ds(k * CHUNK, CHUNK)
                pltpu.async_copy(
                    src_buf.at[idx_s[s].at[sl]], r_ref.at[sl], gsem)

        def finish(b, s):
            r_ref, gsem, ssem = rbufs[b]
            descs = []
            for k in range(BLK):
                sl = pl.ds(k * CHUNK, CHUNK)
                pltpu.make_async_copy(
                    src_buf.at[idx_s[s].at[sl]], r_ref.at[sl], gsem).wait()
                descs.append(pltpu.async_copy(
                    r_ref.at[sl], acc.at[idx_d[s].at[sl]], ssem, add=True))
            for d in descs:
                d.wait()

        @pl.when(wid < EXTRA_TILES)
        def _():
            off = EXTRA_BASE + wid * CHUNK
            csl = pl.ds(0, CHUNK)
            pltpu.sync_copy(src.at[pl.ds(off, CHUNK)], idx_s[0].at[csl])
            pltpu.sync_copy(dst.at[pl.ds(off, CHUNK)], idx_d[0].at[csl])
            pltpu.sync_copy(src_buf.at[idx_s[0].at[csl]], rows0.at[csl])
            pltpu.sync_copy(rows0.at[csl], acc.at[idx_d[0].at[csl]], add=True)

        fire_idx(0, 0)
        fire_idx(1, 1)
        fire_idx(2, 2)
        launch_g(0, 0)

        @pl.loop(0, (N_BLKS - 3) // 4)
        def _(t):
            b0 = 4 * t
            launch_g(1, 1)
            finish(0, 0)
            fire_idx(3, b0 + 3)
            launch_g(0, 2)
            finish(1, 1)
            fire_idx(0, b0 + 4)
            launch_g(1, 3)
            finish(0, 2)
            fire_idx(1, b0 + 5)
            launch_g(0, 0)
            finish(1, 3)
            fire_idx(2, b0 + 6)

        launch_g(1, 1)
        finish(0, 0)
        launch_g(0, 2)
        finish(1, 1)
        finish(0, 2)
        plsc.subcore_barrier()

        zdescs = []

        def restore_zero(k):
            zdescs.append(pltpu.async_copy(
                zbuf, acc.at[pl.ds(rbase + k * WCHUNK, WCHUNK)], ssem0))

        if relu:
            d0 = pltpu.async_copy(acc.at[pl.ds(rbase, BLK_E)], rows0, gsem0)
            d1 = pltpu.async_copy(
                acc.at[pl.ds(rbase + BLK_E, WCHUNK)],
                rows1.at[pl.ds(0, WCHUNK)], gsem1)
            d0.wait()
            for k in range(4):
                restore_zero(k)

            @pl.loop(0, BLK_E)
            def _(r):
                for c in range(HALF // LANES):
                    v = rows0[r, pl.ds(c * LANES, LANES)]
                    rows0[r, pl.ds(c * LANES, LANES)] = jnp.maximum(v, 0.0)

            w0 = pltpu.async_copy(rows0, dst_buf.at[pl.ds(rbase, BLK_E)], ssem1)
            d1.wait()
            restore_zero(4)

            @pl.loop(0, WCHUNK)
            def _(r):
                for c in range(HALF // LANES):
                    v = rows1[r, pl.ds(c * LANES, LANES)]
                    rows1[r, pl.ds(c * LANES, LANES)] = jnp.maximum(v, 0.0)

            w1 = pltpu.async_copy(
                rows1.at[pl.ds(0, WCHUNK)],
                dst_buf.at[pl.ds(rbase + BLK_E, WCHUNK)], ssem1)
            w0.wait()
            w1.wait()
        else:
            w0 = pltpu.async_copy(
                acc.at[pl.ds(rbase, ROWS_PER_TILE)],
                dst_buf.at[pl.ds(rbase, ROWS_PER_TILE)], ssem1)
            w0.wait()
            for k in range(N_WCHUNKS):
                restore_zero(k)
        for d in zdescs:
            d.wait()
        plsc.subcore_barrier()

    @pl.loop(0, WCHUNK)
    def _(r):
        for c in range(HALF // LANES):
            zbuf[r, pl.ds(c * LANES, LANES)] = zeros

    for k in range(N_WCHUNKS):
        pltpu.sync_copy(zbuf, acc.at[pl.ds(rbase + k * WCHUNK, WCHUNK)])
    plsc.subcore_barrier()

    @pl.when(cid == 0)
    def _():
        layer(x_lo, h1_lo, True)
        layer(h1_lo, h2_lo, False)
        layer(h2_lo, o_lo, False)

    @pl.when(cid == 1)
    def _():
        layer(x_hi, h1_hi, True)
        layer(h1_hi, h2_hi, False)
        layer(h2_hi, o_hi, False)


def kernel(x, edge_index):
    src = edge_index[0].astype(jnp.int32)
    dst = edge_index[1].astype(jnp.int32)
    x_lo = x[:, :HALF]
    x_hi = x[:, HALF:]
    *_, o_lo, o_hi = _conv3(x_lo, x_hi, src, dst)
    return jnp.concatenate([o_lo[:N_NODES], o_hi[:N_NODES]], axis=1)


# speedup vs baseline: 1.0737x; 1.0199x over previous
import functools

import jax
import jax.numpy as jnp
from jax import lax
from jax.experimental import pallas as pl
from jax.experimental.pallas import tpu as pltpu
from jax.experimental.pallas import tpu_sc as plsc

N_NODES = 10000
D_FEAT = 128
HALF = D_FEAT // 2
N_EDGES = 320000

N_TILES = 16
CHUNK = 128
N_CHUNKS = N_EDGES // CHUNK
CHUNKS_PER_TILE = N_CHUNKS // N_TILES
EDGES_PER_TILE = CHUNKS_PER_TILE * CHUNK
BLK = 4
BLK_E = BLK * CHUNK
N_BLKS = CHUNKS_PER_TILE // BLK
EXTRA_TILES = N_CHUNKS - N_TILES * CHUNKS_PER_TILE
EXTRA_BASE = N_TILES * EDGES_PER_TILE
N_PAD = 10240
ROWS_PER_TILE = N_PAD // N_TILES
WCHUNK = 128
N_WCHUNKS = ROWS_PER_TILE // WCHUNK
LANES = 16

_mesh = plsc.VectorSubcoreMesh(
    core_axis_name="c", subcore_axis_name="s", num_cores=2
)

_half = jax.ShapeDtypeStruct((N_PAD, HALF), jnp.float32)


@functools.partial(
    pl.kernel,
    out_type=(_half,) * 6,
    mesh=_mesh,
    compiler_params=pltpu.CompilerParams(use_tc_tiling_on_sc=False),
    scratch_types=[
        pltpu.VMEM_SHARED((N_PAD, HALF), jnp.float32),
        pltpu.VMEM((BLK_E, HALF), jnp.float32),
        pltpu.VMEM((BLK_E, HALF), jnp.float32),
        [pltpu.VMEM((BLK_E,), jnp.int32)] * 4,
        [pltpu.VMEM((BLK_E,), jnp.int32)] * 4,
        [pltpu.SemaphoreType.DMA] * 4,
        pltpu.VMEM((WCHUNK, HALF), jnp.float32),
        pltpu.VMEM((CHUNK,), jnp.int32),
        pltpu.VMEM((CHUNK,), jnp.int32),
        pltpu.VMEM((CHUNK, HALF), jnp.float32),
        pltpu.SemaphoreType.DMA,
        pltpu.SemaphoreType.DMA,
        pltpu.SemaphoreType.DMA,
        pltpu.SemaphoreType.DMA,
        pltpu.SemaphoreType.DMA,
    ],
)
def _conv3(x_lo, x_hi, src, dst,
           h1_lo, h1_hi, h2_lo, h2_hi, o_lo, o_hi,
           acc, rows0, rows1, idx_s, idx_d, isem, zbuf,
           eidx_s, eidx_d, erows, esem,
           gsem0, gsem1, ssem0, ssem1):
    cid = lax.axis_index("c")
    wid = lax.axis_index("s")
    ebase = wid * EDGES_PER_TILE
    rbase = wid * ROWS_PER_TILE

    zeros = jnp.zeros((LANES,), jnp.float32)

    def layer(src_buf, dst_buf, relu, first=False, last=False):
        rbufs = ((rows0, gsem0, ssem0), (rows1, gsem1, ssem1))

        def fire_idx(s, blk):
            off = ebase + blk * BLK_E
            pltpu.async_copy(src.at[pl.ds(off, BLK_E)], idx_s[s], isem[s])
            pltpu.async_copy(dst.at[pl.ds(off, BLK_E)], idx_d[s], isem[s])

        def wait_idx(s):
            pltpu.make_async_copy(
                src.at[pl.ds(0, BLK_E)], idx_s[s], isem[s]).wait()
            pltpu.make_async_copy(
                dst.at[pl.ds(0, BLK_E)], idx_d[s], isem[s]).wait()

        def launch_g(b, s):
            r_ref, gsem, _ = rbufs[b]
            wait_idx(s)
            for k in range(BLK):
                sl = pl.ds(k * CHUNK, CHUNK)
                pltpu.async_copy(
                    src_buf.at[idx_s[s].at[sl]], r_ref.at[sl], gsem)

        def finish(b, s):
            r_ref, gsem, ssem = rbufs[b]
            descs = []
            for k in range(BLK):
                sl = pl.ds(k * CHUNK, CHUNK)
                pltpu.make_async_copy(
                    src_buf.at[idx_s[s].at[sl]], r_ref.at[sl], gsem).wait()
                descs.append(pltpu.async_copy(
                    r_ref.at[sl], acc.at[idx_d[s].at[sl]], ssem, add=True))
            for d in descs:
                d.wait()

        @pl.when(wid < EXTRA_TILES)
        def _():
            off = EXTRA_BASE + wid * CHUNK
            pltpu.async_copy(src.at[pl.ds(off, CHUNK)], eidx_s, esem)
            pltpu.async_copy(dst.at[pl.ds(off, CHUNK)], eidx_d, esem)

        if first:
            fire_idx(0, 0)
            fire_idx(1, 1)
            fire_idx(2, 2)
        launch_g(0, 0)

        @pl.when(wid < EXTRA_TILES)
        def _():
            pltpu.make_async_copy(
                src.at[pl.ds(0, CHUNK)], eidx_s, esem).wait()
            pltpu.make_async_copy(
                dst.at[pl.ds(0, CHUNK)], eidx_d, esem).wait()
            pltpu.async_copy(src_buf.at[eidx_s], erows, esem)

        @pl.loop(0, (N_BLKS - 3) // 4)
        def _(t):
            b0 = 4 * t
            launch_g(1, 1)
            finish(0, 0)
            fire_idx(3, b0 + 3)
            launch_g(0, 2)
            finish(1, 1)
            fire_idx(0, b0 + 4)
            launch_g(1, 3)
            finish(0, 2)
            fire_idx(1, b0 + 5)
            launch_g(0, 0)
            finish(1, 3)
            fire_idx(2, b0 + 6)

        launch_g(1, 1)
        finish(0, 0)
        launch_g(0, 2)
        finish(1, 1)
        finish(0, 2)

        @pl.when(wid < EXTRA_TILES)
        def _():
            pltpu.make_async_copy(src_buf.at[eidx_s], erows, esem).wait()
            pltpu.sync_copy(erows, acc.at[eidx_d], add=True)

        if not last:
            fire_idx(0, 0)
            fire_idx(1, 1)
            fire_idx(2, 2)
        plsc.subcore_barrier()

        zdescs = []

        def restore_zero(k):
            zdescs.append(pltpu.async_copy(
                zbuf, acc.at[pl.ds(rbase + k * WCHUNK, WCHUNK)], ssem0))

        if relu:
            d0 = pltpu.async_copy(acc.at[pl.ds(rbase, BLK_E)], rows0, gsem0)
            d1 = pltpu.async_copy(
                acc.at[pl.ds(rbase + BLK_E, WCHUNK)],
                rows1.at[pl.ds(0, WCHUNK)], gsem1)
            d0.wait()
            for k in range(4):
                restore_zero(k)

            @pl.loop(0, BLK_E)
            def _(r):
                for c in range(HALF // LANES):
                    v = rows0[r, pl.ds(c * LANES, LANES)]
                    rows0[r, pl.ds(c * LANES, LANES)] = jnp.maximum(v, 0.0)

            w0 = pltpu.async_copy(rows0, dst_buf.at[pl.ds(rbase, BLK_E)], ssem1)
            d1.wait()
            restore_zero(4)

            @pl.loop(0, WCHUNK)
            def _(r):
                for c in range(HALF // LANES):
                    v = rows1[r, pl.ds(c * LANES, LANES)]
                    rows1[r, pl.ds(c * LANES, LANES)] = jnp.maximum(v, 0.0)

            w1 = pltpu.async_copy(
                rows1.at[pl.ds(0, WCHUNK)],
                dst_buf.at[pl.ds(rbase + BLK_E, WCHUNK)], ssem1)
            w0.wait()
            w1.wait()
        else:
            w0 = pltpu.async_copy(
                acc.at[pl.ds(rbase, ROWS_PER_TILE)],
                dst_buf.at[pl.ds(rbase, ROWS_PER_TILE)], ssem1)
            w0.wait()
            for k in range(N_WCHUNKS):
                restore_zero(k)
        for d in zdescs:
            d.wait()
        plsc.subcore_barrier()

    @pl.loop(0, WCHUNK)
    def _(r):
        for c in range(HALF // LANES):
            zbuf[r, pl.ds(c * LANES, LANES)] = zeros

    for k in range(N_WCHUNKS):
        pltpu.sync_copy(zbuf, acc.at[pl.ds(rbase + k * WCHUNK, WCHUNK)])
    plsc.subcore_barrier()

    @pl.when(cid == 0)
    def _():
        layer(x_lo, h1_lo, True, first=True)
        layer(h1_lo, h2_lo, False)
        layer(h2_lo, o_lo, False, last=True)

    @pl.when(cid == 1)
    def _():
        layer(x_hi, h1_hi, True, first=True)
        layer(h1_hi, h2_hi, False)
        layer(h2_hi, o_hi, False, last=True)


def kernel(x, edge_index):
    src = edge_index[0].astype(jnp.int32)
    dst = edge_index[1].astype(jnp.int32)
    x_lo = x[:, :HALF]
    x_hi = x[:, HALF:]
    *_, o_lo, o_hi = _conv3(x_lo, x_hi, src, dst)
    return jnp.concatenate([o_lo[:N_NODES], o_hi[:N_NODES]], axis=1)
